# baseline (speedup 1.0000x reference)
import functools
import jax
import jax.numpy as jnp
from jax import lax
from jax.experimental import pallas as pl
from jax.experimental.pallas import tpu as pltpu
from jax.experimental.pallas import tpu_sc as plsc

_B, _T, _D = 4, 8192, 1024
_NW = 32
_TPW = _T // _NW
_CHUNK = 32
_NCH = _TPW // _CHUNK


def _sc_body(x_hbm, pos_hbm, out_hbm, pos_v, x_v):
    wid = lax.axis_index("s") * 2 + lax.axis_index("c")
    t0 = wid * _TPW

    def chunk_loop(ci, carry):
        tbase = t0 + ci * _CHUNK
        pltpu.sync_copy(pos_hbm.at[pl.ds(tbase * _D, _CHUNK * _D)], pos_v)

        def b_loop(b, carry2):
            row = b * _T + tbase
            pltpu.sync_copy(x_hbm.at[pl.ds(row * _D, _CHUNK * _D)], x_v)

            @plsc.parallel_loop(0, _CHUNK * _D // 16, unroll=8)
            def add_loop(i):
                sl = pl.ds(i * 16, 16)
                x_v[sl] = x_v[sl] + pos_v[sl]
            pltpu.sync_copy(x_v, out_hbm.at[pl.ds(row * _D, _CHUNK * _D)])
            return carry2

        lax.fori_loop(0, _B, b_loop, 0)
        return carry

    lax.fori_loop(0, _NCH, chunk_loop, 0)


def kernel(x, pos_table):
    B, T, D = x.shape
    mesh = plsc.VectorSubcoreMesh(
        core_axis_name="c", subcore_axis_name="s", num_cores=2, num_subcores=16
    )
    body = functools.partial(
        pl.kernel,
        mesh=mesh,
        out_type=jax.ShapeDtypeStruct((B * T * D,), jnp.float32),
        scratch_types=[
            pltpu.VMEM((_CHUNK * _D,), jnp.float32),
            pltpu.VMEM((_CHUNK * _D,), jnp.float32),
        ],
    )(_sc_body)
    out = body(x.reshape(-1), pos_table.reshape(-1))
    return out.reshape(x.shape)


# speedup vs baseline: 1.3062x; 1.3062x over previous
import functools
import jax
import jax.numpy as jnp
from jax import lax
from jax.experimental import pallas as pl
from jax.experimental.pallas import tpu as pltpu
from jax.experimental.pallas import tpu_sc as plsc

_BT = 1024
_T, _D = 8192, 1024
_NW = 32
_TPW = _T // _NW
_CHUNK = 32
_NCH = _TPW // _CHUNK
_SC_B = 3


def _tc_add(x_ref, pos_ref, out_ref):
    out_ref[...] = x_ref[...] + pos_ref[...]


def _sc_body(x_hbm, pos_hbm, out_hbm, pos_v, x_v):
    wid = lax.axis_index("s") * 2 + lax.axis_index("c")
    t0 = wid * _TPW

    def chunk_loop(ci, carry):
        tbase = t0 + ci * _CHUNK
        pltpu.sync_copy(pos_hbm.at[pl.ds(tbase * _D, _CHUNK * _D)], pos_v)
        row = _SC_B * _T + tbase
        pltpu.sync_copy(x_hbm.at[pl.ds(row * _D, _CHUNK * _D)], x_v)

        @plsc.parallel_loop(0, _CHUNK * _D // 16, unroll=8)
        def add_loop(i):
            sl = pl.ds(i * 16, 16)
            x_v[sl] = x_v[sl] + pos_v[sl]

        pltpu.sync_copy(x_v, out_hbm.at[pl.ds(tbase * _D, _CHUNK * _D)])
        return carry

    lax.fori_loop(0, _NCH, chunk_loop, 0)


def kernel(x, pos_table):
    B, T, D = x.shape
    out_tc = pl.pallas_call(
        _tc_add,
        grid=(T // _BT, B - 1),
        in_specs=[
            pl.BlockSpec((1, _BT, D), lambda t, b: (b, t, 0)),
            pl.BlockSpec((None, _BT, D), lambda t, b: (0, t, 0)),
        ],
        out_specs=pl.BlockSpec((1, _BT, D), lambda t, b: (b, t, 0)),
        out_shape=jax.ShapeDtypeStruct((B - 1, T, D), x.dtype),
    )(x, pos_table[None])

    mesh = plsc.VectorSubcoreMesh(
        core_axis_name="c", subcore_axis_name="s", num_cores=2, num_subcores=16
    )
    sc_call = functools.partial(
        pl.kernel,
        mesh=mesh,
        out_type=jax.ShapeDtypeStruct((T * D,), jnp.float32),
        scratch_types=[
            pltpu.VMEM((_CHUNK * _D,), jnp.float32),
            pltpu.VMEM((_CHUNK * _D,), jnp.float32),
        ],
    )(_sc_body)
    out_sc = sc_call(x.reshape(-1), pos_table.reshape(-1)).reshape(1, T, D)
    return jnp.concatenate([out_tc, out_sc], axis=0)


# speedup vs baseline: 5.1704x; 3.9585x over previous
import jax
import jax.numpy as jnp
from jax.experimental import pallas as pl

_BT = 2048


def _add_kernel(x_ref, pos_ref, out_ref):
    out_ref[...] = x_ref[...] + pos_ref[...]


def kernel(x, pos_table):
    B, T, D = x.shape
    grid = (T // _BT, B)
    return pl.pallas_call(
        _add_kernel,
        grid=grid,
        in_specs=[
            pl.BlockSpec((1, _BT, D), lambda t, b: (b, t, 0)),
            pl.BlockSpec((None, _BT, D), lambda t, b: (0, t, 0)),
        ],
        out_specs=pl.BlockSpec((1, _BT, D), lambda t, b: (b, t, 0)),
        out_shape=jax.ShapeDtypeStruct((B, T, D), x.dtype),
    )(x, pos_table[None])
